# Initial kernel scaffold; baseline (speedup 1.0000x reference)
import jax
import jax.numpy as jnp
from jax.experimental import pallas as pl


def kernel(h, edge_index, edge_weight, edge_attr, W_lin1, W_m1, b_m1, W_m2, b_m2, W_lin2, b_lin2, W_out, b_out):
    raise NotImplementedError("write your pallas kernel here")


# speedup vs baseline: 1.3365x; 1.3365x over previous
import functools
import math
from math import pi as PI

import jax
import jax.numpy as jnp
from jax import lax
from jax.experimental import pallas as pl
from jax.experimental.pallas import tpu as pltpu
from jax.experimental.pallas import tpu_sc as plsc

CUTOFF = 10.0
SHIFT = math.log(2.0)

NC = 2
NS = 16
LANES = 16
NW = NC * NS

CK = 80


def _ssp(x):
    return jnp.maximum(x, 0.0) + jnp.log1p(jnp.exp(-jnp.abs(x))) - SHIFT


def _node_matmul(h, w):
    n, _ = h.shape
    m = w.shape[1]

    def body(h_ref, w_ref, o_ref):
        o_ref[...] = jnp.dot(h_ref[...], w_ref[...],
                             preferred_element_type=jnp.float32)

    return pl.pallas_call(
        body,
        out_shape=jax.ShapeDtypeStruct((n, m), jnp.float32),
    )(h, w)


def _edge_mlp(edge_attr, edge_weight, W_m1, b_m1, W_m2, b_m2):
    E, NG = edge_attr.shape
    NF = W_m1.shape[1]
    BE = 3200
    grid = (E // BE,)
    ew = edge_weight.reshape(E, 1)

    def body(ea_ref, ew_ref, w1_ref, b1_ref, w2_ref, b2_ref, o_ref):
        hid = jnp.dot(ea_ref[...], w1_ref[...],
                      preferred_element_type=jnp.float32) + b1_ref[...]
        hid = _ssp(hid)
        w = jnp.dot(hid, w2_ref[...],
                    preferred_element_type=jnp.float32) + b2_ref[...]
        c = 0.5 * (jnp.cos(ew_ref[...] * (PI / CUTOFF)) + 1.0)
        o_ref[...] = w * c

    return pl.pallas_call(
        body,
        grid=grid,
        in_specs=[
            pl.BlockSpec((BE, NG), lambda i: (i, 0)),
            pl.BlockSpec((BE, 1), lambda i: (i, 0)),
            pl.BlockSpec((NG, NF), lambda i: (0, 0)),
            pl.BlockSpec((1, NF), lambda i: (0, 0)),
            pl.BlockSpec((NF, NF), lambda i: (0, 0)),
            pl.BlockSpec((1, NF), lambda i: (0, 0)),
        ],
        out_specs=pl.BlockSpec((BE, NF), lambda i: (i, 0)),
        out_shape=jax.ShapeDtypeStruct((E, NF), jnp.float32),
    )(edge_attr, ew, W_m1, b_m1.reshape(1, NF), W_m2, b_m2.reshape(1, NF))


def _sc_gather_scatter(h1, w_e, src, dst):
    N, HC = h1.shape
    E = src.shape[0]
    epw = E // NW
    nchunks = epw // CK
    npad = ((N + 8 * NS - 1) // (8 * NS)) * (8 * NS)
    rps = npad // NS

    mesh = plsc.VectorSubcoreMesh(core_axis_name="c", subcore_axis_name="s")

    @functools.partial(
        pl.kernel,
        out_type=jax.ShapeDtypeStruct((NC, npad, HC), jnp.float32),
        mesh=mesh,
        scratch_types=[
            pltpu.VMEM((CK,), jnp.int32),
            pltpu.VMEM((CK,), jnp.int32),
            pltpu.VMEM((CK, HC), jnp.float32),
            pltpu.VMEM((CK, HC), jnp.float32),
            pltpu.VMEM_SHARED((npad, HC), jnp.float32),
        ],
    )
    def k(zeros_hbm, h1_hbm, we_hbm, src_hbm, dst_hbm, out_hbm,
          sidx_v, didx_v, rows_v, w_v, agg_sh):
        c = lax.axis_index("c")
        s = lax.axis_index("s")
        wid = s * NC + c

        pltpu.sync_copy(zeros_hbm, agg_sh.at[pl.ds(s * rps, rps)])
        plsc.subcore_barrier()

        base0 = wid * epw

        @pl.loop(0, nchunks)
        def _(i):
            base = base0 + i * CK
            pltpu.sync_copy(src_hbm.at[pl.ds(base, CK)], sidx_v)
            pltpu.sync_copy(dst_hbm.at[pl.ds(base, CK)], didx_v)
            pltpu.sync_copy(h1_hbm.at[sidx_v], rows_v)
            pltpu.sync_copy(we_hbm.at[pl.ds(base, CK)], w_v)

            @pl.loop(0, CK)
            def _(r):
                for cc in range(0, HC, LANES):
                    sl = (r, pl.ds(cc, LANES))
                    rows_v.at[sl][...] = rows_v.at[sl][...] * w_v.at[sl][...]

            pltpu.sync_copy(rows_v, agg_sh.at[didx_v], add=True)

        plsc.subcore_barrier()
        pltpu.sync_copy(agg_sh.at[pl.ds(s * rps, rps)],
                        out_hbm.at[c, pl.ds(s * rps, rps)])

    zeros = jnp.zeros((rps, HC), jnp.float32)
    return k(zeros, h1, w_e, src, dst)[:, :N, :]


def _tail(partials, W_lin2, b_lin2, W_out, b_out):
    _, N, NF = partials.shape
    HC = W_lin2.shape[1]

    def body(p_ref, w1_ref, b1_ref, w2_ref, b2_ref, o_ref):
        agg = p_ref[0] + p_ref[1]
        h2 = jnp.dot(agg, w1_ref[...],
                     preferred_element_type=jnp.float32) + b1_ref[...]
        h3 = _ssp(h2)
        o_ref[...] = jnp.dot(h3, w2_ref[...],
                             preferred_element_type=jnp.float32) + b2_ref[...]

    return pl.pallas_call(
        body,
        out_shape=jax.ShapeDtypeStruct((N, HC), jnp.float32),
    )(partials, W_lin2, b_lin2.reshape(1, HC), W_out, b_out.reshape(1, HC))


def kernel(h, edge_index, edge_weight, edge_attr,
           W_lin1, W_m1, b_m1, W_m2, b_m2, W_lin2, b_lin2, W_out, b_out):
    src = edge_index[1].astype(jnp.int32)
    dst = edge_index[0].astype(jnp.int32)

    h1 = _node_matmul(h, W_lin1)
    w_e = _edge_mlp(edge_attr, edge_weight, W_m1, b_m1, W_m2, b_m2)
    partials = _sc_gather_scatter(h1, w_e, src, dst)
    return _tail(partials, W_lin2, b_lin2, W_out, b_out)


# speedup vs baseline: 1.6540x; 1.2375x over previous
import dataclasses
import functools
import math
from math import pi as PI

import jax
import jax.numpy as jnp
from jax import lax
from jax.experimental import pallas as pl
from jax.experimental.pallas import tpu as pltpu
from jax.experimental.pallas import tpu_sc as plsc

CUTOFF = 10.0
SHIFT = math.log(2.0)

NC = 2
NS = 16
LANES = 16
NW = NC * NS

CK = 128


def _ssp(x):
    return jnp.maximum(x, 0.0) + jnp.log1p(jnp.exp(-jnp.abs(x))) - SHIFT


def _node_matmul(h, w):
    n, _ = h.shape
    m = w.shape[1]

    def body(h_ref, w_ref, o_ref):
        o_ref[...] = jnp.dot(h_ref[...], w_ref[...],
                             preferred_element_type=jnp.float32)

    return pl.pallas_call(
        body,
        out_shape=jax.ShapeDtypeStruct((n, m), jnp.float32),
    )(h, w)


def _cutoff(ew2d):
    def body(ew_ref, c_ref):
        c_ref[...] = 0.5 * (jnp.cos(ew_ref[...] * (PI / CUTOFF)) + 1.0)

    return pl.pallas_call(
        body,
        out_shape=jax.ShapeDtypeStruct(ew2d.shape, jnp.float32),
    )(ew2d)


def _edge_mlp(edge_attr, W_m1, b_m1, W_m2, b_m2):
    E, NG = edge_attr.shape
    NF = W_m1.shape[1]
    BE = 3200
    grid = (E // BE,)

    def body(ea_ref, w1_ref, b1_ref, w2_ref, b2_ref, o_ref):
        hid = jnp.dot(ea_ref[...], w1_ref[...],
                      preferred_element_type=jnp.float32) + b1_ref[...]
        hid = _ssp(hid)
        o_ref[...] = jnp.dot(hid, w2_ref[...],
                             preferred_element_type=jnp.float32) + b2_ref[...]

    return pl.pallas_call(
        body,
        grid=grid,
        in_specs=[
            pl.BlockSpec((BE, NG), lambda i: (i, 0)),
            pl.BlockSpec((NG, NF), lambda i: (0, 0)),
            pl.BlockSpec((1, NF), lambda i: (0, 0)),
            pl.BlockSpec((NF, NF), lambda i: (0, 0)),
            pl.BlockSpec((1, NF), lambda i: (0, 0)),
        ],
        out_specs=pl.BlockSpec((BE, NF), lambda i: (i, 0)),
        out_shape=jax.ShapeDtypeStruct((E, NF), jnp.float32),
    )(edge_attr, W_m1, b_m1.reshape(1, NF), W_m2, b_m2.reshape(1, NF))


def _sc_gather_scatter(h1, m_e, c2d, ei3):
    N, HC = h1.shape
    nrows = ei3.shape[1]
    rpw = nrows // NW
    nextra = nrows - rpw * NW
    npad = ((N + 8 * NS - 1) // (8 * NS)) * (8 * NS)
    rps = npad // NS

    mesh = plsc.VectorSubcoreMesh(core_axis_name="c", subcore_axis_name="s")
    cp = pltpu.CompilerParams()
    if "needs_layout_passes" in pltpu.CompilerParams.__dataclass_fields__:
        cp = dataclasses.replace(cp, needs_layout_passes=False)

    @functools.partial(
        pl.kernel,
        out_type=jax.ShapeDtypeStruct((NC, npad, HC), jnp.float32),
        mesh=mesh,
        compiler_params=cp,
        scratch_types=[
            pltpu.VMEM((CK,), jnp.int32),
            pltpu.VMEM((CK,), jnp.int32),
            pltpu.VMEM((CK,), jnp.float32),
            pltpu.VMEM((CK, HC), jnp.float32),
            pltpu.VMEM((CK, HC), jnp.float32),
            pltpu.VMEM_SHARED((npad, HC), jnp.float32),
        ],
    )
    def k(zeros_hbm, h1_hbm, me_hbm, c_hbm, ei_hbm, out_hbm,
          sidx_v, didx_v, c_v, rows_v, w_v, agg_sh):
        c = lax.axis_index("c")
        s = lax.axis_index("s")
        wid = s * NC + c

        pltpu.sync_copy(zeros_hbm, agg_sh.at[pl.ds(s * rps, rps)])
        plsc.subcore_barrier()

        def do_row(row):
            pltpu.sync_copy(ei_hbm.at[1, row], sidx_v)
            pltpu.sync_copy(ei_hbm.at[0, row], didx_v)
            pltpu.sync_copy(h1_hbm.at[sidx_v], rows_v)
            pltpu.sync_copy(me_hbm.at[pl.ds(row * CK, CK)], w_v)
            pltpu.sync_copy(c_hbm.at[row], c_v)

            @pl.loop(0, CK)
            def _(r):
                cs = plsc.load_gather(
                    c_v, [jnp.full((LANES,), r, jnp.int32)])
                for cc in range(0, HC, LANES):
                    sl = (r, pl.ds(cc, LANES))
                    rows_v.at[sl][...] = (rows_v.at[sl][...]
                                          * (w_v.at[sl][...] * cs))

            pltpu.sync_copy(rows_v, agg_sh.at[didx_v], add=True)

        @pl.loop(0, rpw)
        def _(i):
            do_row(wid * rpw + i)

        if nextra:
            @pl.when(wid < nextra)
            def _():
                do_row(NW * rpw + wid)

        plsc.subcore_barrier()
        pltpu.sync_copy(agg_sh.at[pl.ds(s * rps, rps)],
                        out_hbm.at[c, pl.ds(s * rps, rps)])

    zeros = jnp.zeros((rps, HC), jnp.float32)
    return k(zeros, h1, m_e, c2d, ei3)[:, :N, :]


def _tail(partials, W_lin2, b_lin2, W_out, b_out):
    _, N, NF = partials.shape
    HC = W_lin2.shape[1]

    def body(p_ref, w1_ref, b1_ref, w2_ref, b2_ref, o_ref):
        agg = p_ref[0] + p_ref[1]
        h2 = jnp.dot(agg, w1_ref[...],
                     preferred_element_type=jnp.float32) + b1_ref[...]
        h3 = _ssp(h2)
        o_ref[...] = jnp.dot(h3, w2_ref[...],
                             preferred_element_type=jnp.float32) + b2_ref[...]

    return pl.pallas_call(
        body,
        out_shape=jax.ShapeDtypeStruct((N, HC), jnp.float32),
    )(partials, W_lin2, b_lin2.reshape(1, HC), W_out, b_out.reshape(1, HC))


def kernel(h, edge_index, edge_weight, edge_attr,
           W_lin1, W_m1, b_m1, W_m2, b_m2, W_lin2, b_lin2, W_out, b_out):
    E = edge_weight.shape[0]
    ei3 = edge_index.astype(jnp.int32).reshape(2, E // CK, CK)
    ew2d = edge_weight.reshape(E // CK, CK)

    h1 = _node_matmul(h, W_lin1)
    c2d = _cutoff(ew2d)
    m_e = _edge_mlp(edge_attr, W_m1, b_m1, W_m2, b_m2)
    partials = _sc_gather_scatter(h1, m_e, c2d, ei3)
    return _tail(partials, W_lin2, b_lin2, W_out, b_out)


# speedup vs baseline: 2.2255x; 1.3455x over previous
import dataclasses
import functools
import math
from math import pi as PI

import jax
import jax.numpy as jnp
from jax import lax
from jax.experimental import pallas as pl
from jax.experimental.pallas import tpu as pltpu
from jax.experimental.pallas import tpu_sc as plsc

CUTOFF = 10.0
SHIFT = math.log(2.0)

NC = 2
NS = 16
LANES = 16
NW = NC * NS

CK = 128


def _ssp(x):
    return jnp.maximum(x, 0.0) + jnp.log1p(jnp.exp(-jnp.abs(x))) - SHIFT


def _node_matmul(h, w):
    n, _ = h.shape
    m = w.shape[1]

    def body(h_ref, w_ref, o_ref):
        o_ref[...] = jnp.dot(h_ref[...], w_ref[...],
                             preferred_element_type=jnp.float32)

    return pl.pallas_call(
        body,
        out_shape=jax.ShapeDtypeStruct((n, m), jnp.float32),
    )(h, w)


def _cutoff(ew2d):
    def body(ew_ref, c_ref):
        c_ref[...] = 0.5 * (jnp.cos(ew_ref[...] * (PI / CUTOFF)) + 1.0)

    return pl.pallas_call(
        body,
        out_shape=jax.ShapeDtypeStruct(ew2d.shape, jnp.float32),
    )(ew2d)


def _edge_mlp(edge_attr, W_m1, b_m1, W_m2, b_m2):
    E, NG = edge_attr.shape
    NF = W_m1.shape[1]
    BE = 3200
    grid = (E // BE,)

    def body(ea_ref, w1_ref, b1_ref, w2_ref, b2_ref, o_ref):
        hid = jnp.dot(ea_ref[...], w1_ref[...],
                      preferred_element_type=jnp.float32) + b1_ref[...]
        hid = _ssp(hid)
        o_ref[...] = jnp.dot(hid, w2_ref[...],
                             preferred_element_type=jnp.float32) + b2_ref[...]

    return pl.pallas_call(
        body,
        grid=grid,
        in_specs=[
            pl.BlockSpec((BE, NG), lambda i: (i, 0)),
            pl.BlockSpec((NG, NF), lambda i: (0, 0)),
            pl.BlockSpec((1, NF), lambda i: (0, 0)),
            pl.BlockSpec((NF, NF), lambda i: (0, 0)),
            pl.BlockSpec((1, NF), lambda i: (0, 0)),
        ],
        out_specs=pl.BlockSpec((BE, NF), lambda i: (i, 0)),
        out_shape=jax.ShapeDtypeStruct((E, NF), jnp.float32),
    )(edge_attr, W_m1, b_m1.reshape(1, NF), W_m2, b_m2.reshape(1, NF))


def _sc_gather_scatter(h1, m_e, c2d, ei3):
    N, HC = h1.shape
    nrows = ei3.shape[1]
    rpw = nrows // NW
    nextra = nrows - rpw * NW
    npad = ((N + 8 * NS - 1) // (8 * NS)) * (8 * NS)
    rps = npad // NS

    mesh = plsc.VectorSubcoreMesh(core_axis_name="c", subcore_axis_name="s")
    cp = pltpu.CompilerParams()
    if "needs_layout_passes" in pltpu.CompilerParams.__dataclass_fields__:
        cp = dataclasses.replace(cp, needs_layout_passes=False)

    DI = 3
    DG = 2
    STEP = 6
    CB = 64
    nslot = 2 * rpw
    assert nslot % STEP == 0

    @functools.partial(
        pl.kernel,
        out_type=jax.ShapeDtypeStruct((NC, npad, HC), jnp.float32),
        mesh=mesh,
        compiler_params=cp,
        scratch_types=(
            [pltpu.VMEM((CB,), jnp.int32) for _ in range(DI)]
            + [pltpu.VMEM((CB,), jnp.int32) for _ in range(DI)]
            + [pltpu.VMEM((CB,), jnp.float32) for _ in range(DI)]
            + [pltpu.VMEM((CB, HC), jnp.float32) for _ in range(DI)]
            + [pltpu.VMEM((CB, HC), jnp.float32) for _ in range(DG)]
            + [pltpu.SemaphoreType.DMA for _ in range(2 * DI + 2 * DG)]
            + [pltpu.VMEM_SHARED((npad, HC), jnp.float32)]
        ),
    )
    def k(zeros_hbm, h1_hbm, me_hbm, c_hbm, ei_hbm, out_hbm, *scr):
        sidx = scr[0:DI]
        didx = scr[DI:2 * DI]
        c_v = scr[2 * DI:3 * DI]
        w_v = scr[3 * DI:4 * DI]
        rows = scr[4 * DI:4 * DI + DG]
        sems = scr[4 * DI + DG:4 * DI + DG + 2 * DI + 2 * DG]
        sem_i = sems[0:DI]
        sem_w = sems[DI:2 * DI]
        sem_g = sems[2 * DI:2 * DI + DG]
        sem_s = sems[2 * DI + DG:2 * DI + 2 * DG]
        agg_sh = scr[-1]

        c = lax.axis_index("c")
        s = lax.axis_index("s")
        wid = s * NC + c

        pltpu.sync_copy(zeros_hbm, agg_sh.at[pl.ds(s * rps, rps)])
        plsc.subcore_barrier()

        base2 = wid * nslot

        def start_loads(t, m):
            row = (base2 + t) // 2
            off = (base2 + t) % 2 * CB
            pltpu.async_copy(ei_hbm.at[1, row, pl.ds(off, CB)], sidx[m],
                             sem_i[m])
            pltpu.async_copy(ei_hbm.at[0, row, pl.ds(off, CB)], didx[m],
                             sem_i[m])
            pltpu.async_copy(me_hbm.at[pl.ds(row * CK + off, CB)], w_v[m],
                             sem_w[m])
            pltpu.async_copy(c_hbm.at[row, pl.ds(off, CB)], c_v[m], sem_w[m])

        def wait_loads_idx(m):
            pltpu.make_async_copy(ei_hbm.at[1, 0, pl.ds(0, CB)], sidx[m],
                                  sem_i[m]).wait()
            pltpu.make_async_copy(ei_hbm.at[0, 0, pl.ds(0, CB)], didx[m],
                                  sem_i[m]).wait()

        def wait_loads_wc(m):
            pltpu.make_async_copy(me_hbm.at[pl.ds(0, CB)], w_v[m],
                                  sem_w[m]).wait()
            pltpu.make_async_copy(c_hbm.at[0, pl.ds(0, CB)], c_v[m],
                                  sem_w[m]).wait()

        def start_gather(m, b):
            pltpu.async_copy(h1_hbm.at[sidx[m]], rows[b], sem_g[b])

        def wait_gather(b):
            pltpu.make_async_copy(h1_hbm.at[pl.ds(0, CB)], rows[b],
                                  sem_g[b]).wait()

        def start_scatter(m, b):
            pltpu.async_copy(rows[b], agg_sh.at[didx[m]], sem_s[b],
                             add=True)

        def wait_scatter(b):
            pltpu.make_async_copy(rows[b], agg_sh.at[pl.ds(0, CB)],
                                  sem_s[b]).wait()

        def mult(m, b):
            @pl.loop(0, CB, step=2)
            def _(r0):
                for dr in range(2):
                    r = r0 + dr
                    cs = plsc.load_gather(
                        c_v[m], [jnp.full((LANES,), r, jnp.int32)])
                    for cc in range(0, HC, LANES):
                        sl = (r, pl.ds(cc, LANES))
                        rows[b].at[sl][...] = (rows[b].at[sl][...]
                                               * (w_v[m].at[sl][...] * cs))

        start_loads(0, 0)
        start_loads(1, 1)
        wait_loads_idx(0)
        start_gather(0, 0)

        @pl.loop(0, nslot, step=STEP)
        def _(g):
            for j in range(STEP):
                b = j % DG
                m = j % DI
                nb = (j + 1) % DG
                nm = (j + 1) % DI
                i = g + j

                @pl.when(i + 1 < nslot)
                def _():
                    wait_loads_idx(nm)

                    @pl.when(i >= 1)
                    def _():
                        wait_scatter(nb)

                    start_gather(nm, nb)

                @pl.when(i + 2 < nslot)
                def _():
                    start_loads(i + 2, (j + 2) % DI)

                wait_gather(b)
                wait_loads_wc(m)
                mult(m, b)
                start_scatter(m, b)

        wait_scatter(0)
        wait_scatter(1)

        if nextra:
            @pl.when(wid < nextra)
            def _():
                row = NW * rpw + wid
                for h in range(2):
                    off = h * CB
                    pltpu.sync_copy(ei_hbm.at[1, row, pl.ds(off, CB)],
                                    sidx[0])
                    pltpu.sync_copy(ei_hbm.at[0, row, pl.ds(off, CB)],
                                    didx[0])
                    pltpu.sync_copy(h1_hbm.at[sidx[0]], rows[0])
                    pltpu.sync_copy(me_hbm.at[pl.ds(row * CK + off, CB)],
                                    w_v[0])
                    pltpu.sync_copy(c_hbm.at[row, pl.ds(off, CB)], c_v[0])
                    mult(0, 0)
                    pltpu.sync_copy(rows[0], agg_sh.at[didx[0]], add=True)

        plsc.subcore_barrier()
        pltpu.sync_copy(agg_sh.at[pl.ds(s * rps, rps)],
                        out_hbm.at[c, pl.ds(s * rps, rps)])

    zeros = jnp.zeros((rps, HC), jnp.float32)
    return k(zeros, h1, m_e, c2d, ei3)[:, :N, :]


def _tail(partials, W_lin2, b_lin2, W_out, b_out):
    _, N, NF = partials.shape
    HC = W_lin2.shape[1]

    def body(p_ref, w1_ref, b1_ref, w2_ref, b2_ref, o_ref):
        agg = p_ref[0] + p_ref[1]
        h2 = jnp.dot(agg, w1_ref[...],
                     preferred_element_type=jnp.float32) + b1_ref[...]
        h3 = _ssp(h2)
        o_ref[...] = jnp.dot(h3, w2_ref[...],
                             preferred_element_type=jnp.float32) + b2_ref[...]

    return pl.pallas_call(
        body,
        out_shape=jax.ShapeDtypeStruct((N, HC), jnp.float32),
    )(partials, W_lin2, b_lin2.reshape(1, HC), W_out, b_out.reshape(1, HC))


def kernel(h, edge_index, edge_weight, edge_attr,
           W_lin1, W_m1, b_m1, W_m2, b_m2, W_lin2, b_lin2, W_out, b_out):
    E = edge_weight.shape[0]
    ei3 = edge_index.astype(jnp.int32).reshape(2, E // CK, CK)
    ew2d = edge_weight.reshape(E // CK, CK)

    h1 = _node_matmul(h, W_lin1)
    c2d = _cutoff(ew2d)
    m_e = _edge_mlp(edge_attr, W_m1, b_m1, W_m2, b_m2)
    partials = _sc_gather_scatter(h1, m_e, c2d, ei3)
    return _tail(partials, W_lin2, b_lin2, W_out, b_out)


# speedup vs baseline: 3.6086x; 1.6215x over previous
import dataclasses
import functools
import math
from math import pi as PI

import jax
import jax.numpy as jnp
from jax import lax
from jax.experimental import pallas as pl
from jax.experimental.pallas import tpu as pltpu
from jax.experimental.pallas import tpu_sc as plsc

CUTOFF = 10.0
SHIFT = math.log(2.0)

NC = 2
NS = 16
LANES = 16
NW = NC * NS

CK = 128


def _ssp(x):
    return jnp.maximum(x, 0.0) + jnp.log1p(jnp.exp(-jnp.abs(x))) - SHIFT


def _node_matmul(h, w):
    n, _ = h.shape
    m = w.shape[1]

    def body(h_ref, w_ref, o_ref):
        o_ref[...] = jnp.dot(h_ref[...], w_ref[...],
                             preferred_element_type=jnp.float32)

    return pl.pallas_call(
        body,
        out_shape=jax.ShapeDtypeStruct((n, m), jnp.float32),
    )(h, w)


def _cutoff(ew2d):
    def body(ew_ref, c_ref):
        c_ref[...] = 0.5 * (jnp.cos(ew_ref[...] * (PI / CUTOFF)) + 1.0)

    return pl.pallas_call(
        body,
        out_shape=jax.ShapeDtypeStruct(ew2d.shape, jnp.float32),
    )(ew2d)


def _edge_mlp(edge_attr, W_m1, b_m1, W_m2, b_m2):
    E, NG = edge_attr.shape
    NF = W_m1.shape[1]
    BE = 3200
    grid = (E // BE,)

    def body(ea_ref, w1_ref, b1_ref, w2_ref, b2_ref, o_ref):
        hid = jnp.dot(ea_ref[...], w1_ref[...],
                      preferred_element_type=jnp.float32) + b1_ref[...]
        hid = _ssp(hid)
        o_ref[...] = jnp.dot(hid, w2_ref[...],
                             preferred_element_type=jnp.float32) + b2_ref[...]

    return pl.pallas_call(
        body,
        grid=grid,
        in_specs=[
            pl.BlockSpec((BE, NG), lambda i: (i, 0)),
            pl.BlockSpec((NG, NF), lambda i: (0, 0)),
            pl.BlockSpec((1, NF), lambda i: (0, 0)),
            pl.BlockSpec((NF, NF), lambda i: (0, 0)),
            pl.BlockSpec((1, NF), lambda i: (0, 0)),
        ],
        out_specs=pl.BlockSpec((BE, NF), lambda i: (i, 0)),
        out_shape=jax.ShapeDtypeStruct((E, NF), jnp.float32),
    )(edge_attr, W_m1, b_m1.reshape(1, NF), W_m2, b_m2.reshape(1, NF))


def _sc_gather_scatter(h1, m_e, c2d, ei3):
    N, HC = h1.shape
    nrows = ei3.shape[1]
    rpw = nrows // NW
    nextra = nrows - rpw * NW
    npad = ((N + 8 * NS - 1) // (8 * NS)) * (8 * NS)
    rps = npad // NS

    mesh = plsc.VectorSubcoreMesh(core_axis_name="c", subcore_axis_name="s")
    cp = pltpu.CompilerParams()
    if "needs_layout_passes" in pltpu.CompilerParams.__dataclass_fields__:
        cp = dataclasses.replace(cp, needs_layout_passes=False)

    DI = 3
    DG = 2
    STEP = 6
    CB = 64
    nslot = 2 * rpw
    assert nslot % STEP == 0

    @functools.partial(
        pl.kernel,
        out_type=jax.ShapeDtypeStruct((NC, npad, HC), jnp.float32),
        mesh=mesh,
        compiler_params=cp,
        scratch_types=(
            [pltpu.VMEM((CB,), jnp.int32) for _ in range(DI)]
            + [pltpu.VMEM((CB,), jnp.int32) for _ in range(DI)]
            + [pltpu.VMEM((CB,), jnp.float32) for _ in range(DI)]
            + [pltpu.VMEM((CB, HC), jnp.float32) for _ in range(DI)]
            + [pltpu.VMEM((CB, HC), jnp.float32) for _ in range(DG)]
            + [pltpu.SemaphoreType.DMA for _ in range(2 * DI + 2 * DG)]
            + [pltpu.VMEM_SHARED((npad, HC), jnp.float32)]
        ),
    )
    def k(zeros_hbm, h1_hbm, me_hbm, c_hbm, ei_hbm, out_hbm, *scr):
        sidx = scr[0:DI]
        didx = scr[DI:2 * DI]
        c_v = scr[2 * DI:3 * DI]
        w_v = scr[3 * DI:4 * DI]
        rows = scr[4 * DI:4 * DI + DG]
        sems = scr[4 * DI + DG:4 * DI + DG + 2 * DI + 2 * DG]
        sem_i = sems[0:DI]
        sem_w = sems[DI:2 * DI]
        sem_g = sems[2 * DI:2 * DI + DG]
        sem_s = sems[2 * DI + DG:2 * DI + 2 * DG]
        agg_sh = scr[-1]

        c = lax.axis_index("c")
        s = lax.axis_index("s")
        wid = s * NC + c

        pltpu.sync_copy(zeros_hbm, agg_sh.at[pl.ds(s * rps, rps)])
        plsc.subcore_barrier()

        base2 = wid * nslot

        def start_loads(t, m):
            row = (base2 + t) // 2
            off = (base2 + t) % 2 * CB
            pltpu.async_copy(ei_hbm.at[1, row, pl.ds(off, CB)], sidx[m],
                             sem_i[m])
            pltpu.async_copy(ei_hbm.at[0, row, pl.ds(off, CB)], didx[m],
                             sem_i[m])
            pltpu.async_copy(me_hbm.at[pl.ds(row * CK + off, CB)], w_v[m],
                             sem_w[m])
            pltpu.async_copy(c_hbm.at[row, pl.ds(off, CB)], c_v[m], sem_w[m])

        def wait_loads_idx(m):
            pltpu.make_async_copy(ei_hbm.at[1, 0, pl.ds(0, CB)], sidx[m],
                                  sem_i[m]).wait()
            pltpu.make_async_copy(ei_hbm.at[0, 0, pl.ds(0, CB)], didx[m],
                                  sem_i[m]).wait()

        def wait_loads_wc(m):
            pltpu.make_async_copy(me_hbm.at[pl.ds(0, CB)], w_v[m],
                                  sem_w[m]).wait()
            pltpu.make_async_copy(c_hbm.at[0, pl.ds(0, CB)], c_v[m],
                                  sem_w[m]).wait()

        def start_gather(m, b):
            pltpu.async_copy(h1_hbm.at[sidx[m]], rows[b], sem_g[b])

        def wait_gather(b):
            pltpu.make_async_copy(h1_hbm.at[pl.ds(0, CB)], rows[b],
                                  sem_g[b]).wait()

        def start_scatter(m, b):
            pltpu.async_copy(rows[b], agg_sh.at[didx[m]], sem_s[b],
                             add=True)

        def wait_scatter(b):
            pltpu.make_async_copy(rows[b], agg_sh.at[pl.ds(0, CB)],
                                  sem_s[b]).wait()

        def mult(m, b):
            @plsc.parallel_loop(0, CB, step=1, unroll=8)
            def _(r):
                cs = plsc.load_gather(
                    c_v[m], [jnp.full((LANES,), r, jnp.int32)])
                for cc in range(0, HC, LANES):
                    sl = (r, pl.ds(cc, LANES))
                    rows[b].at[sl][...] = (rows[b].at[sl][...]
                                           * (w_v[m].at[sl][...] * cs))

        start_loads(0, 0)
        start_loads(1, 1)
        wait_loads_idx(0)
        start_gather(0, 0)

        @pl.loop(0, nslot, step=STEP)
        def _(g):
            for j in range(STEP):
                b = j % DG
                m = j % DI
                nb = (j + 1) % DG
                nm = (j + 1) % DI
                i = g + j

                @pl.when(i + 1 < nslot)
                def _():
                    wait_loads_idx(nm)

                    @pl.when(i >= 1)
                    def _():
                        wait_scatter(nb)

                    start_gather(nm, nb)

                @pl.when(i + 2 < nslot)
                def _():
                    start_loads(i + 2, (j + 2) % DI)

                wait_gather(b)
                wait_loads_wc(m)
                mult(m, b)
                start_scatter(m, b)

        wait_scatter(0)
        wait_scatter(1)

        if nextra:
            @pl.when(wid < nextra)
            def _():
                row = NW * rpw + wid
                for h in range(2):
                    off = h * CB
                    pltpu.sync_copy(ei_hbm.at[1, row, pl.ds(off, CB)],
                                    sidx[0])
                    pltpu.sync_copy(ei_hbm.at[0, row, pl.ds(off, CB)],
                                    didx[0])
                    pltpu.sync_copy(h1_hbm.at[sidx[0]], rows[0])
                    pltpu.sync_copy(me_hbm.at[pl.ds(row * CK + off, CB)],
                                    w_v[0])
                    pltpu.sync_copy(c_hbm.at[row, pl.ds(off, CB)], c_v[0])
                    mult(0, 0)
                    pltpu.sync_copy(rows[0], agg_sh.at[didx[0]], add=True)

        plsc.subcore_barrier()
        pltpu.sync_copy(agg_sh.at[pl.ds(s * rps, rps)],
                        out_hbm.at[c, pl.ds(s * rps, rps)])

    zeros = jnp.zeros((rps, HC), jnp.float32)
    return k(zeros, h1, m_e, c2d, ei3)[:, :N, :]


def _tail(partials, W_lin2, b_lin2, W_out, b_out):
    _, N, NF = partials.shape
    HC = W_lin2.shape[1]

    def body(p_ref, w1_ref, b1_ref, w2_ref, b2_ref, o_ref):
        agg = p_ref[0] + p_ref[1]
        h2 = jnp.dot(agg, w1_ref[...],
                     preferred_element_type=jnp.float32) + b1_ref[...]
        h3 = _ssp(h2)
        o_ref[...] = jnp.dot(h3, w2_ref[...],
                             preferred_element_type=jnp.float32) + b2_ref[...]

    return pl.pallas_call(
        body,
        out_shape=jax.ShapeDtypeStruct((N, HC), jnp.float32),
    )(partials, W_lin2, b_lin2.reshape(1, HC), W_out, b_out.reshape(1, HC))


def kernel(h, edge_index, edge_weight, edge_attr,
           W_lin1, W_m1, b_m1, W_m2, b_m2, W_lin2, b_lin2, W_out, b_out):
    E = edge_weight.shape[0]
    ei3 = edge_index.astype(jnp.int32).reshape(2, E // CK, CK)
    ew2d = edge_weight.reshape(E // CK, CK)

    h1 = _node_matmul(h, W_lin1)
    c2d = _cutoff(ew2d)
    m_e = _edge_mlp(edge_attr, W_m1, b_m1, W_m2, b_m2)
    partials = _sc_gather_scatter(h1, m_e, c2d, ei3)
    return _tail(partials, W_lin2, b_lin2, W_out, b_out)


# speedup vs baseline: 3.6183x; 1.0027x over previous
import dataclasses
import functools
import math
from math import pi as PI

import jax
import jax.numpy as jnp
from jax import lax
from jax.experimental import pallas as pl
from jax.experimental.pallas import tpu as pltpu
from jax.experimental.pallas import tpu_sc as plsc

CUTOFF = 10.0
SHIFT = math.log(2.0)

NC = 2
NS = 16
LANES = 16
NW = NC * NS

CK = 128


def _ssp(x):
    return jnp.maximum(x, 0.0) + jnp.log1p(jnp.exp(-jnp.abs(x))) - SHIFT


def _node_matmul(h, w):
    n, _ = h.shape
    m = w.shape[1]

    def body(h_ref, w_ref, o_ref):
        o_ref[...] = jnp.dot(h_ref[...], w_ref[...],
                             preferred_element_type=jnp.float32)

    return pl.pallas_call(
        body,
        out_shape=jax.ShapeDtypeStruct((n, m), jnp.float32),
    )(h, w)


def _repack_idx(edge_index):
    E = edge_index.shape[1]

    def body(ei_ref, o_ref):
        o_ref[...] = ei_ref[...].reshape(2, E // CK, CK)

    return pl.pallas_call(
        body,
        out_shape=jax.ShapeDtypeStruct((2, E // CK, CK), jnp.int32),
    )(edge_index)


def _cutoff(ew2d):
    def body(ew_ref, c_ref):
        c_ref[...] = 0.5 * (jnp.cos(ew_ref[...] * (PI / CUTOFF)) + 1.0)

    return pl.pallas_call(
        body,
        out_shape=jax.ShapeDtypeStruct(ew2d.shape, jnp.float32),
    )(ew2d)


def _edge_mlp(edge_attr, W_m1, b_m1, W_m2, b_m2):
    E, NG = edge_attr.shape
    NF = W_m1.shape[1]
    BE = 3200
    grid = (E // BE,)

    def body(ea_ref, w1_ref, b1_ref, w2_ref, b2_ref, o_ref):
        hid = jnp.dot(ea_ref[...], w1_ref[...],
                      preferred_element_type=jnp.float32) + b1_ref[...]
        hid = _ssp(hid)
        o_ref[...] = jnp.dot(hid, w2_ref[...],
                             preferred_element_type=jnp.float32) + b2_ref[...]

    return pl.pallas_call(
        body,
        grid=grid,
        in_specs=[
            pl.BlockSpec((BE, NG), lambda i: (i, 0)),
            pl.BlockSpec((NG, NF), lambda i: (0, 0)),
            pl.BlockSpec((1, NF), lambda i: (0, 0)),
            pl.BlockSpec((NF, NF), lambda i: (0, 0)),
            pl.BlockSpec((1, NF), lambda i: (0, 0)),
        ],
        out_specs=pl.BlockSpec((BE, NF), lambda i: (i, 0)),
        out_shape=jax.ShapeDtypeStruct((E, NF), jnp.float32),
    )(edge_attr, W_m1, b_m1.reshape(1, NF), W_m2, b_m2.reshape(1, NF))


def _sc_gather_scatter(h1, m_e, c2d, ei3):
    N, HC = h1.shape
    nrows = ei3.shape[1]
    rpw = nrows // NW
    nextra = nrows - rpw * NW
    npad = ((N + 8 * NS - 1) // (8 * NS)) * (8 * NS)
    rps = npad // NS

    mesh = plsc.VectorSubcoreMesh(core_axis_name="c", subcore_axis_name="s")
    cp = pltpu.CompilerParams()
    if "needs_layout_passes" in pltpu.CompilerParams.__dataclass_fields__:
        cp = dataclasses.replace(cp, needs_layout_passes=False)

    DI = 3
    DG = 2
    STEP = 6
    CB = 64
    nslot = 2 * rpw
    assert nslot % STEP == 0

    @functools.partial(
        pl.kernel,
        out_type=jax.ShapeDtypeStruct((NC, npad, HC), jnp.float32),
        mesh=mesh,
        compiler_params=cp,
        scratch_types=(
            [pltpu.VMEM((CB,), jnp.int32) for _ in range(DI)]
            + [pltpu.VMEM((CB,), jnp.int32) for _ in range(DI)]
            + [pltpu.VMEM((CB,), jnp.float32) for _ in range(DI)]
            + [pltpu.VMEM((CB, HC), jnp.float32) for _ in range(DI)]
            + [pltpu.VMEM((CB, HC), jnp.float32) for _ in range(DG)]
            + [pltpu.SemaphoreType.DMA for _ in range(2 * DI + 2 * DG)]
            + [pltpu.VMEM_SHARED((npad, HC), jnp.float32)]
        ),
    )
    def k(zeros_hbm, h1_hbm, me_hbm, c_hbm, ei_hbm, out_hbm, *scr):
        sidx = scr[0:DI]
        didx = scr[DI:2 * DI]
        c_v = scr[2 * DI:3 * DI]
        w_v = scr[3 * DI:4 * DI]
        rows = scr[4 * DI:4 * DI + DG]
        sems = scr[4 * DI + DG:4 * DI + DG + 2 * DI + 2 * DG]
        sem_i = sems[0:DI]
        sem_w = sems[DI:2 * DI]
        sem_g = sems[2 * DI:2 * DI + DG]
        sem_s = sems[2 * DI + DG:2 * DI + 2 * DG]
        agg_sh = scr[-1]

        c = lax.axis_index("c")
        s = lax.axis_index("s")
        wid = s * NC + c

        pltpu.sync_copy(zeros_hbm, agg_sh.at[pl.ds(s * rps, rps)])
        plsc.subcore_barrier()

        base2 = wid * nslot

        def start_loads(t, m):
            row = (base2 + t) // 2
            off = (base2 + t) % 2 * CB
            pltpu.async_copy(ei_hbm.at[1, row, pl.ds(off, CB)], sidx[m],
                             sem_i[m])
            pltpu.async_copy(ei_hbm.at[0, row, pl.ds(off, CB)], didx[m],
                             sem_i[m])
            pltpu.async_copy(me_hbm.at[pl.ds(row * CK + off, CB)], w_v[m],
                             sem_w[m])
            pltpu.async_copy(c_hbm.at[row, pl.ds(off, CB)], c_v[m], sem_w[m])

        def wait_loads_idx(m):
            pltpu.make_async_copy(ei_hbm.at[1, 0, pl.ds(0, CB)], sidx[m],
                                  sem_i[m]).wait()
            pltpu.make_async_copy(ei_hbm.at[0, 0, pl.ds(0, CB)], didx[m],
                                  sem_i[m]).wait()

        def wait_loads_wc(m):
            pltpu.make_async_copy(me_hbm.at[pl.ds(0, CB)], w_v[m],
                                  sem_w[m]).wait()
            pltpu.make_async_copy(c_hbm.at[0, pl.ds(0, CB)], c_v[m],
                                  sem_w[m]).wait()

        def start_gather(m, b):
            pltpu.async_copy(h1_hbm.at[sidx[m]], rows[b], sem_g[b])

        def wait_gather(b):
            pltpu.make_async_copy(h1_hbm.at[pl.ds(0, CB)], rows[b],
                                  sem_g[b]).wait()

        def start_scatter(m, b):
            pltpu.async_copy(rows[b], agg_sh.at[didx[m]], sem_s[b],
                             add=True)

        def wait_scatter(b):
            pltpu.make_async_copy(rows[b], agg_sh.at[pl.ds(0, CB)],
                                  sem_s[b]).wait()

        def mult(m, b):
            @plsc.parallel_loop(0, CB, step=1, unroll=8)
            def _(r):
                cs = plsc.load_gather(
                    c_v[m], [jnp.full((LANES,), r, jnp.int32)])
                for cc in range(0, HC, LANES):
                    sl = (r, pl.ds(cc, LANES))
                    rows[b].at[sl][...] = (rows[b].at[sl][...]
                                           * (w_v[m].at[sl][...] * cs))

        start_loads(0, 0)
        start_loads(1, 1)
        wait_loads_idx(0)
        start_gather(0, 0)

        @pl.loop(0, nslot, step=STEP)
        def _(g):
            for j in range(STEP):
                b = j % DG
                m = j % DI
                nb = (j + 1) % DG
                nm = (j + 1) % DI
                i = g + j

                @pl.when(i + 1 < nslot)
                def _():
                    wait_loads_idx(nm)

                    @pl.when(i >= 1)
                    def _():
                        wait_scatter(nb)

                    start_gather(nm, nb)

                @pl.when(i + 2 < nslot)
                def _():
                    start_loads(i + 2, (j + 2) % DI)

                wait_gather(b)
                wait_loads_wc(m)
                mult(m, b)
                start_scatter(m, b)

        wait_scatter(0)
        wait_scatter(1)

        if nextra:
            @pl.when(wid < nextra)
            def _():
                row = NW * rpw + wid
                for h in range(2):
                    off = h * CB
                    pltpu.sync_copy(ei_hbm.at[1, row, pl.ds(off, CB)],
                                    sidx[0])
                    pltpu.sync_copy(ei_hbm.at[0, row, pl.ds(off, CB)],
                                    didx[0])
                    pltpu.sync_copy(h1_hbm.at[sidx[0]], rows[0])
                    pltpu.sync_copy(me_hbm.at[pl.ds(row * CK + off, CB)],
                                    w_v[0])
                    pltpu.sync_copy(c_hbm.at[row, pl.ds(off, CB)], c_v[0])
                    mult(0, 0)
                    pltpu.sync_copy(rows[0], agg_sh.at[didx[0]], add=True)

        plsc.subcore_barrier()
        pltpu.sync_copy(agg_sh.at[pl.ds(s * rps, rps)],
                        out_hbm.at[c, pl.ds(s * rps, rps)])

    zeros = jnp.zeros((rps, HC), jnp.float32)
    return k(zeros, h1, m_e, c2d, ei3)[:, :N, :]


def _tail(partials, W_lin2, b_lin2, W_out, b_out):
    _, N, NF = partials.shape
    HC = W_lin2.shape[1]

    def body(p_ref, w1_ref, b1_ref, w2_ref, b2_ref, o_ref):
        agg = p_ref[0] + p_ref[1]
        h2 = jnp.dot(agg, w1_ref[...],
                     preferred_element_type=jnp.float32) + b1_ref[...]
        h3 = _ssp(h2)
        o_ref[...] = jnp.dot(h3, w2_ref[...],
                             preferred_element_type=jnp.float32) + b2_ref[...]

    return pl.pallas_call(
        body,
        out_shape=jax.ShapeDtypeStruct((N, HC), jnp.float32),
    )(partials, W_lin2, b_lin2.reshape(1, HC), W_out, b_out.reshape(1, HC))


def kernel(h, edge_index, edge_weight, edge_attr,
           W_lin1, W_m1, b_m1, W_m2, b_m2, W_lin2, b_lin2, W_out, b_out):
    E = edge_weight.shape[0]
    ei3 = _repack_idx(edge_index.astype(jnp.int32))
    ew2d = edge_weight.reshape(E // CK, CK)

    h1 = _node_matmul(h, W_lin1)
    c2d = _cutoff(ew2d)
    m_e = _edge_mlp(edge_attr, W_m1, b_m1, W_m2, b_m2)
    partials = _sc_gather_scatter(h1, m_e, c2d, ei3)
    return _tail(partials, W_lin2, b_lin2, W_out, b_out)


# speedup vs baseline: 4.6582x; 1.2874x over previous
import dataclasses
import functools
import math
from math import pi as PI

import jax
import jax.numpy as jnp
from jax import lax
from jax.experimental import pallas as pl
from jax.experimental.pallas import tpu as pltpu
from jax.experimental.pallas import tpu_sc as plsc

CUTOFF = 10.0
SHIFT = math.log(2.0)

NC = 2
NS = 16
LANES = 16
NW = NC * NS

CK = 128


def _ssp(x):
    return jnp.maximum(x, 0.0) + jnp.log1p(jnp.exp(-jnp.abs(x))) - SHIFT


def _node_matmul(h, w):
    n, _ = h.shape
    m = w.shape[1]

    def body(h_ref, w_ref, o_ref):
        o_ref[...] = jnp.dot(h_ref[...], w_ref[...],
                             preferred_element_type=jnp.float32)

    return pl.pallas_call(
        body,
        out_shape=jax.ShapeDtypeStruct((n, m), jnp.float32),
    )(h, w)


def _repack_idx(edge_index):
    E = edge_index.shape[1]

    def body(ei_ref, o_ref):
        o_ref[...] = ei_ref[...].reshape(2, E // CK, CK)

    return pl.pallas_call(
        body,
        out_shape=jax.ShapeDtypeStruct((2, E // CK, CK), jnp.int32),
    )(edge_index)


def _cutoff(ew2d):
    def body(ew_ref, c_ref):
        c_ref[...] = 0.5 * (jnp.cos(ew_ref[...] * (PI / CUTOFF)) + 1.0)

    return pl.pallas_call(
        body,
        out_shape=jax.ShapeDtypeStruct(ew2d.shape, jnp.float32),
    )(ew2d)


def _edge_mlp(ea_t, W_m1, b_m1, W_m2, b_m2):
    NG, E = ea_t.shape
    NF = W_m1.shape[1]
    BE = 3200
    grid = (E // BE,)

    def body(ea_ref, w1_ref, b1_ref, w2_ref, b2_ref, o_ref):
        hid = lax.dot_general(
            ea_ref[...], w1_ref[...], (((0,), (0,)), ((), ())),
            preferred_element_type=jnp.float32) + b1_ref[...]
        hid = _ssp(hid)
        o_ref[...] = jnp.dot(hid, w2_ref[...],
                             preferred_element_type=jnp.float32) + b2_ref[...]

    return pl.pallas_call(
        body,
        grid=grid,
        in_specs=[
            pl.BlockSpec((NG, BE), lambda i: (0, i)),
            pl.BlockSpec((NG, NF), lambda i: (0, 0)),
            pl.BlockSpec((1, NF), lambda i: (0, 0)),
            pl.BlockSpec((NF, NF), lambda i: (0, 0)),
            pl.BlockSpec((1, NF), lambda i: (0, 0)),
        ],
        out_specs=pl.BlockSpec((BE, NF), lambda i: (i, 0)),
        out_shape=jax.ShapeDtypeStruct((E, NF), jnp.float32),
    )(ea_t, W_m1, b_m1.reshape(1, NF), W_m2, b_m2.reshape(1, NF))


def _sc_gather_scatter(h1, m_e, c2d, ei3):
    N, HC = h1.shape
    nrows = ei3.shape[1]
    rpw = nrows // NW
    nextra = nrows - rpw * NW
    npad = ((N + 8 * NS - 1) // (8 * NS)) * (8 * NS)
    rps = npad // NS

    mesh = plsc.VectorSubcoreMesh(core_axis_name="c", subcore_axis_name="s")
    cp = pltpu.CompilerParams()
    if "needs_layout_passes" in pltpu.CompilerParams.__dataclass_fields__:
        cp = dataclasses.replace(cp, needs_layout_passes=False)

    DI = 3
    DG = 2
    STEP = 6
    CB = 64
    nslot = 2 * rpw
    assert nslot % STEP == 0

    @functools.partial(
        pl.kernel,
        out_type=jax.ShapeDtypeStruct((NC, npad, HC), jnp.float32),
        mesh=mesh,
        compiler_params=cp,
        scratch_types=(
            [pltpu.VMEM((CB,), jnp.int32) for _ in range(DI)]
            + [pltpu.VMEM((CB,), jnp.int32) for _ in range(DI)]
            + [pltpu.VMEM((CB,), jnp.float32) for _ in range(DI)]
            + [pltpu.VMEM((CB, HC), jnp.float32) for _ in range(DI)]
            + [pltpu.VMEM((CB, HC), jnp.float32) for _ in range(DG)]
            + [pltpu.SemaphoreType.DMA for _ in range(2 * DI + 2 * DG)]
            + [pltpu.VMEM_SHARED((npad, HC), jnp.float32)]
        ),
    )
    def k(zeros_hbm, h1_hbm, me_hbm, c_hbm, ei_hbm, out_hbm, *scr):
        sidx = scr[0:DI]
        didx = scr[DI:2 * DI]
        c_v = scr[2 * DI:3 * DI]
        w_v = scr[3 * DI:4 * DI]
        rows = scr[4 * DI:4 * DI + DG]
        sems = scr[4 * DI + DG:4 * DI + DG + 2 * DI + 2 * DG]
        sem_i = sems[0:DI]
        sem_w = sems[DI:2 * DI]
        sem_g = sems[2 * DI:2 * DI + DG]
        sem_s = sems[2 * DI + DG:2 * DI + 2 * DG]
        agg_sh = scr[-1]

        c = lax.axis_index("c")
        s = lax.axis_index("s")
        wid = s * NC + c

        pltpu.sync_copy(zeros_hbm, agg_sh.at[pl.ds(s * rps, rps)])
        plsc.subcore_barrier()

        base2 = wid * nslot

        def start_loads(t, m):
            row = (base2 + t) // 2
            off = (base2 + t) % 2 * CB
            pltpu.async_copy(ei_hbm.at[1, row, pl.ds(off, CB)], sidx[m],
                             sem_i[m])
            pltpu.async_copy(ei_hbm.at[0, row, pl.ds(off, CB)], didx[m],
                             sem_i[m])
            pltpu.async_copy(me_hbm.at[pl.ds(row * CK + off, CB)], w_v[m],
                             sem_w[m])
            pltpu.async_copy(c_hbm.at[row, pl.ds(off, CB)], c_v[m], sem_w[m])

        def wait_loads_idx(m):
            pltpu.make_async_copy(ei_hbm.at[1, 0, pl.ds(0, CB)], sidx[m],
                                  sem_i[m]).wait()
            pltpu.make_async_copy(ei_hbm.at[0, 0, pl.ds(0, CB)], didx[m],
                                  sem_i[m]).wait()

        def wait_loads_wc(m):
            pltpu.make_async_copy(me_hbm.at[pl.ds(0, CB)], w_v[m],
                                  sem_w[m]).wait()
            pltpu.make_async_copy(c_hbm.at[0, pl.ds(0, CB)], c_v[m],
                                  sem_w[m]).wait()

        def start_gather(m, b):
            pltpu.async_copy(h1_hbm.at[sidx[m]], rows[b], sem_g[b])

        def wait_gather(b):
            pltpu.make_async_copy(h1_hbm.at[pl.ds(0, CB)], rows[b],
                                  sem_g[b]).wait()

        def start_scatter(m, b):
            pltpu.async_copy(rows[b], agg_sh.at[didx[m]], sem_s[b],
                             add=True)

        def wait_scatter(b):
            pltpu.make_async_copy(rows[b], agg_sh.at[pl.ds(0, CB)],
                                  sem_s[b]).wait()

        def mult(m, b):
            @plsc.parallel_loop(0, CB, step=1, unroll=8)
            def _(r):
                cs = plsc.load_gather(
                    c_v[m], [jnp.full((LANES,), r, jnp.int32)])
                for cc in range(0, HC, LANES):
                    sl = (r, pl.ds(cc, LANES))
                    rows[b].at[sl][...] = (rows[b].at[sl][...]
                                           * (w_v[m].at[sl][...] * cs))

        start_loads(0, 0)
        start_loads(1, 1)
        wait_loads_idx(0)
        start_gather(0, 0)

        @pl.loop(0, nslot, step=STEP)
        def _(g):
            for j in range(STEP):
                b = j % DG
                m = j % DI
                nb = (j + 1) % DG
                nm = (j + 1) % DI
                i = g + j

                @pl.when(i + 1 < nslot)
                def _():
                    wait_loads_idx(nm)

                    @pl.when(i >= 1)
                    def _():
                        wait_scatter(nb)

                    start_gather(nm, nb)

                @pl.when(i + 2 < nslot)
                def _():
                    start_loads(i + 2, (j + 2) % DI)

                wait_gather(b)
                wait_loads_wc(m)
                mult(m, b)
                start_scatter(m, b)

        wait_scatter(0)
        wait_scatter(1)

        if nextra:
            @pl.when(wid < nextra)
            def _():
                row = NW * rpw + wid
                for h in range(2):
                    off = h * CB
                    pltpu.sync_copy(ei_hbm.at[1, row, pl.ds(off, CB)],
                                    sidx[0])
                    pltpu.sync_copy(ei_hbm.at[0, row, pl.ds(off, CB)],
                                    didx[0])
                    pltpu.sync_copy(h1_hbm.at[sidx[0]], rows[0])
                    pltpu.sync_copy(me_hbm.at[pl.ds(row * CK + off, CB)],
                                    w_v[0])
                    pltpu.sync_copy(c_hbm.at[row, pl.ds(off, CB)], c_v[0])
                    mult(0, 0)
                    pltpu.sync_copy(rows[0], agg_sh.at[didx[0]], add=True)

        plsc.subcore_barrier()
        pltpu.sync_copy(agg_sh.at[pl.ds(s * rps, rps)],
                        out_hbm.at[c, pl.ds(s * rps, rps)])

    zeros = jnp.zeros((rps, HC), jnp.float32)
    return k(zeros, h1, m_e, c2d, ei3)[:, :N, :]


def _tail(partials, W_lin2, b_lin2, W_out, b_out):
    _, N, NF = partials.shape
    HC = W_lin2.shape[1]

    def body(p_ref, w1_ref, b1_ref, w2_ref, b2_ref, o_ref):
        agg = p_ref[0] + p_ref[1]
        h2 = jnp.dot(agg, w1_ref[...],
                     preferred_element_type=jnp.float32) + b1_ref[...]
        h3 = _ssp(h2)
        o_ref[...] = jnp.dot(h3, w2_ref[...],
                             preferred_element_type=jnp.float32) + b2_ref[...]

    return pl.pallas_call(
        body,
        out_shape=jax.ShapeDtypeStruct((N, HC), jnp.float32),
    )(partials, W_lin2, b_lin2.reshape(1, HC), W_out, b_out.reshape(1, HC))


def kernel(h, edge_index, edge_weight, edge_attr,
           W_lin1, W_m1, b_m1, W_m2, b_m2, W_lin2, b_lin2, W_out, b_out):
    E = edge_weight.shape[0]
    ei3 = _repack_idx(edge_index.astype(jnp.int32))
    ew2d = edge_weight.reshape(E // CK, CK)

    h1 = _node_matmul(h, W_lin1)
    c2d = _cutoff(ew2d)
    m_e = _edge_mlp(edge_attr.T, W_m1, b_m1, W_m2, b_m2)
    partials = _sc_gather_scatter(h1, m_e, c2d, ei3)
    return _tail(partials, W_lin2, b_lin2, W_out, b_out)


# speedup vs baseline: 4.8649x; 1.0444x over previous
import dataclasses
import functools
import math
from math import pi as PI

import jax
import jax.numpy as jnp
from jax import lax
from jax.experimental import pallas as pl
from jax.experimental.pallas import tpu as pltpu
from jax.experimental.pallas import tpu_sc as plsc

CUTOFF = 10.0
SHIFT = math.log(2.0)

NC = 2
NS = 16
LANES = 16
NW = NC * NS

CK = 128


def _sp(x):
    return jnp.where(x > 20.0, x - SHIFT, jnp.log(0.5 + 0.5 * jnp.exp(x)))


def _node_matmul(h, w):
    n, _ = h.shape
    m = w.shape[1]

    def body(h_ref, w_ref, o_ref):
        o_ref[...] = jnp.dot(h_ref[...], w_ref[...],
                             preferred_element_type=jnp.float32)

    return pl.pallas_call(
        body,
        out_shape=jax.ShapeDtypeStruct((n, m), jnp.float32),
    )(h, w)


def _repack_idx(edge_index):
    E = edge_index.shape[1]

    def body(ei_ref, o_ref):
        o_ref[...] = ei_ref[...].reshape(2, E // CK, CK)

    return pl.pallas_call(
        body,
        out_shape=jax.ShapeDtypeStruct((2, E // CK, CK), jnp.int32),
    )(edge_index)


def _cutoff(ew2d):
    def body(ew_ref, c_ref):
        c_ref[...] = 0.5 * (jnp.cos(ew_ref[...] * (PI / CUTOFF)) + 1.0)

    return pl.pallas_call(
        body,
        out_shape=jax.ShapeDtypeStruct(ew2d.shape, jnp.float32),
    )(ew2d)


def _edge_mlp(ea_t, W_m1, b_m1, W_m2, b_m2):
    NG, E = ea_t.shape
    NF = W_m1.shape[1]
    BE = 3200
    grid = (E // BE,)

    def body(ea_ref, w1_ref, b1_ref, w2_ref, b2_ref, o_ref):
        hid = lax.dot_general(
            ea_ref[...], w1_ref[...], (((0,), (0,)), ((), ())),
            preferred_element_type=jnp.float32) + b1_ref[...]
        hid = _sp(hid)
        o_ref[...] = jnp.dot(hid, w2_ref[...],
                             preferred_element_type=jnp.float32) + b2_ref[...]

    return pl.pallas_call(
        body,
        grid=grid,
        in_specs=[
            pl.BlockSpec((NG, BE), lambda i: (0, i)),
            pl.BlockSpec((NG, NF), lambda i: (0, 0)),
            pl.BlockSpec((1, NF), lambda i: (0, 0)),
            pl.BlockSpec((NF, NF), lambda i: (0, 0)),
            pl.BlockSpec((1, NF), lambda i: (0, 0)),
        ],
        out_specs=pl.BlockSpec((BE, NF), lambda i: (i, 0)),
        out_shape=jax.ShapeDtypeStruct((E, NF), jnp.float32),
    )(ea_t, W_m1, b_m1.reshape(1, NF), W_m2, b_m2.reshape(1, NF))


def _sc_gather_scatter(h1, m_e, c2d, ei3):
    N, HC = h1.shape
    nrows = ei3.shape[1]
    rpw = nrows // NW
    nextra = nrows - rpw * NW
    npad = ((N + 8 * NS - 1) // (8 * NS)) * (8 * NS)
    rps = npad // NS

    mesh = plsc.VectorSubcoreMesh(core_axis_name="c", subcore_axis_name="s")
    cp = pltpu.CompilerParams()
    if "needs_layout_passes" in pltpu.CompilerParams.__dataclass_fields__:
        cp = dataclasses.replace(cp, needs_layout_passes=False)

    DI = 3
    DG = 2
    STEP = 6
    CB = 64
    nslot = 2 * rpw
    assert nslot % STEP == 0

    @functools.partial(
        pl.kernel,
        out_type=jax.ShapeDtypeStruct((NC, npad, HC), jnp.float32),
        mesh=mesh,
        compiler_params=cp,
        scratch_types=(
            [pltpu.VMEM((CB,), jnp.int32) for _ in range(DI)]
            + [pltpu.VMEM((CB,), jnp.int32) for _ in range(DI)]
            + [pltpu.VMEM((CB,), jnp.float32) for _ in range(DI)]
            + [pltpu.VMEM((CB, HC), jnp.float32) for _ in range(DI)]
            + [pltpu.VMEM((CB, HC), jnp.float32) for _ in range(DG)]
            + [pltpu.SemaphoreType.DMA for _ in range(2 * DI + 2 * DG)]
            + [pltpu.VMEM_SHARED((npad, HC), jnp.float32)]
        ),
    )
    def k(zeros_hbm, h1_hbm, me_hbm, c_hbm, ei_hbm, out_hbm, *scr):
        sidx = scr[0:DI]
        didx = scr[DI:2 * DI]
        c_v = scr[2 * DI:3 * DI]
        w_v = scr[3 * DI:4 * DI]
        rows = scr[4 * DI:4 * DI + DG]
        sems = scr[4 * DI + DG:4 * DI + DG + 2 * DI + 2 * DG]
        sem_i = sems[0:DI]
        sem_w = sems[DI:2 * DI]
        sem_g = sems[2 * DI:2 * DI + DG]
        sem_s = sems[2 * DI + DG:2 * DI + 2 * DG]
        agg_sh = scr[-1]

        c = lax.axis_index("c")
        s = lax.axis_index("s")
        wid = s * NC + c

        pltpu.sync_copy(zeros_hbm, agg_sh.at[pl.ds(s * rps, rps)])
        plsc.subcore_barrier()

        base2 = wid * nslot

        def start_loads(t, m):
            row = (base2 + t) // 2
            off = (base2 + t) % 2 * CB
            pltpu.async_copy(ei_hbm.at[1, row, pl.ds(off, CB)], sidx[m],
                             sem_i[m])
            pltpu.async_copy(ei_hbm.at[0, row, pl.ds(off, CB)], didx[m],
                             sem_i[m])
            pltpu.async_copy(me_hbm.at[pl.ds(row * CK + off, CB)], w_v[m],
                             sem_w[m])
            pltpu.async_copy(c_hbm.at[row, pl.ds(off, CB)], c_v[m], sem_w[m])

        def wait_loads_idx(m):
            pltpu.make_async_copy(ei_hbm.at[1, 0, pl.ds(0, CB)], sidx[m],
                                  sem_i[m]).wait()
            pltpu.make_async_copy(ei_hbm.at[0, 0, pl.ds(0, CB)], didx[m],
                                  sem_i[m]).wait()

        def wait_loads_wc(m):
            pltpu.make_async_copy(me_hbm.at[pl.ds(0, CB)], w_v[m],
                                  sem_w[m]).wait()
            pltpu.make_async_copy(c_hbm.at[0, pl.ds(0, CB)], c_v[m],
                                  sem_w[m]).wait()

        def start_gather(m, b):
            pltpu.async_copy(h1_hbm.at[sidx[m]], rows[b], sem_g[b])

        def wait_gather(b):
            pltpu.make_async_copy(h1_hbm.at[pl.ds(0, CB)], rows[b],
                                  sem_g[b]).wait()

        def start_scatter(m, b):
            pltpu.async_copy(rows[b], agg_sh.at[didx[m]], sem_s[b],
                             add=True)

        def wait_scatter(b):
            pltpu.make_async_copy(rows[b], agg_sh.at[pl.ds(0, CB)],
                                  sem_s[b]).wait()

        def mult(m, b):
            @plsc.parallel_loop(0, CB, step=1, unroll=8)
            def _(r):
                cs = plsc.load_gather(
                    c_v[m], [jnp.full((LANES,), r, jnp.int32)])
                for cc in range(0, HC, LANES):
                    sl = (r, pl.ds(cc, LANES))
                    rows[b].at[sl][...] = (rows[b].at[sl][...]
                                           * (w_v[m].at[sl][...] * cs))

        start_loads(0, 0)
        start_loads(1, 1)
        wait_loads_idx(0)
        start_gather(0, 0)

        @pl.loop(0, nslot, step=STEP)
        def _(g):
            for j in range(STEP):
                b = j % DG
                m = j % DI
                nb = (j + 1) % DG
                nm = (j + 1) % DI
                i = g + j

                @pl.when(i + 1 < nslot)
                def _():
                    wait_loads_idx(nm)

                    @pl.when(i >= 1)
                    def _():
                        wait_scatter(nb)

                    start_gather(nm, nb)

                @pl.when(i + 2 < nslot)
                def _():
                    start_loads(i + 2, (j + 2) % DI)

                wait_gather(b)
                wait_loads_wc(m)
                mult(m, b)
                start_scatter(m, b)

        wait_scatter(0)
        wait_scatter(1)

        if nextra:
            @pl.when(wid < nextra)
            def _():
                row = NW * rpw + wid
                for h in range(2):
                    off = h * CB
                    pltpu.sync_copy(ei_hbm.at[1, row, pl.ds(off, CB)],
                                    sidx[0])
                    pltpu.sync_copy(ei_hbm.at[0, row, pl.ds(off, CB)],
                                    didx[0])
                    pltpu.sync_copy(h1_hbm.at[sidx[0]], rows[0])
                    pltpu.sync_copy(me_hbm.at[pl.ds(row * CK + off, CB)],
                                    w_v[0])
                    pltpu.sync_copy(c_hbm.at[row, pl.ds(off, CB)], c_v[0])
                    mult(0, 0)
                    pltpu.sync_copy(rows[0], agg_sh.at[didx[0]], add=True)

        plsc.subcore_barrier()
        pltpu.sync_copy(agg_sh.at[pl.ds(s * rps, rps)],
                        out_hbm.at[c, pl.ds(s * rps, rps)])

    zeros = jnp.zeros((rps, HC), jnp.float32)
    return k(zeros, h1, m_e, c2d, ei3)[:, :N, :]


def _tail(partials, W_lin2, b_lin2, W_out, b_out):
    _, N, NF = partials.shape
    HC = W_lin2.shape[1]

    def body(p_ref, w1_ref, b1_ref, w2_ref, b2_ref, o_ref):
        agg = p_ref[0] + p_ref[1]
        h2 = jnp.dot(agg, w1_ref[...],
                     preferred_element_type=jnp.float32) + b1_ref[...]
        h3 = _sp(h2)
        o_ref[...] = jnp.dot(h3, w2_ref[...],
                             preferred_element_type=jnp.float32) + b2_ref[...]

    return pl.pallas_call(
        body,
        out_shape=jax.ShapeDtypeStruct((N, HC), jnp.float32),
    )(partials, W_lin2, b_lin2.reshape(1, HC), W_out, b_out.reshape(1, HC))


def kernel(h, edge_index, edge_weight, edge_attr,
           W_lin1, W_m1, b_m1, W_m2, b_m2, W_lin2, b_lin2, W_out, b_out):
    E = edge_weight.shape[0]
    ei3 = _repack_idx(edge_index.astype(jnp.int32))
    ew2d = edge_weight.reshape(E // CK, CK)

    h1 = _node_matmul(h, W_lin1)
    c2d = _cutoff(ew2d)
    m_e = _edge_mlp(edge_attr.T, W_m1, b_m1, W_m2, b_m2)
    partials = _sc_gather_scatter(h1, m_e, c2d, ei3)
    return _tail(partials, W_lin2, b_lin2, W_out, b_out)


# speedup vs baseline: 5.1239x; 1.0532x over previous
import dataclasses
import functools
import math
from math import pi as PI

import jax
import jax.numpy as jnp
from jax import lax
from jax.experimental import pallas as pl
from jax.experimental.pallas import tpu as pltpu
from jax.experimental.pallas import tpu_sc as plsc

CUTOFF = 10.0
SHIFT = math.log(2.0)

NC = 2
NS = 16
LANES = 16
NW = NC * NS

CK = 128


def _sp(x):
    return jnp.where(x > 20.0, x - SHIFT, jnp.log(0.5 + 0.5 * jnp.exp(x)))


def _node_matmul(h, w):
    n, _ = h.shape
    m = w.shape[1]

    def body(h_ref, w_ref, o_ref):
        o_ref[...] = jnp.dot(h_ref[...], w_ref[...],
                             preferred_element_type=jnp.float32)

    return pl.pallas_call(
        body,
        out_shape=jax.ShapeDtypeStruct((n, m), jnp.float32),
    )(h, w)


def _repack_idx(edge_index):
    E = edge_index.shape[1]

    def body(ei_ref, o_ref):
        o_ref[...] = ei_ref[...].reshape(2, E // CK, CK)

    return pl.pallas_call(
        body,
        out_shape=jax.ShapeDtypeStruct((2, E // CK, CK), jnp.int32),
    )(edge_index)


def _cutoff(ew2d):
    def body(ew_ref, c_ref):
        c_ref[...] = 0.5 * (jnp.cos(ew_ref[...] * (PI / CUTOFF)) + 1.0)

    return pl.pallas_call(
        body,
        out_shape=jax.ShapeDtypeStruct(ew2d.shape, jnp.float32),
    )(ew2d)


def _edge_mlp(ea_t, W_m1, b_m1, W_m2, b_m2, erange):
    NG, E = ea_t.shape
    NF = W_m1.shape[1]
    BE = 3200
    e0, e1 = erange
    grid = ((e1 - e0) // BE,)
    blk0 = e0 // BE

    def body(ea_ref, w1_ref, b1_ref, w2_ref, b2_ref, o_ref):
        hid = lax.dot_general(
            ea_ref[...], w1_ref[...], (((0,), (0,)), ((), ())),
            preferred_element_type=jnp.float32) + b1_ref[...]
        hid = _sp(hid)
        o_ref[...] = jnp.dot(hid, w2_ref[...],
                             preferred_element_type=jnp.float32) + b2_ref[...]

    return pl.pallas_call(
        body,
        grid=grid,
        in_specs=[
            pl.BlockSpec((NG, BE), lambda i: (0, i + blk0)),
            pl.BlockSpec((NG, NF), lambda i: (0, 0)),
            pl.BlockSpec((1, NF), lambda i: (0, 0)),
            pl.BlockSpec((NF, NF), lambda i: (0, 0)),
            pl.BlockSpec((1, NF), lambda i: (0, 0)),
        ],
        out_specs=pl.BlockSpec((BE, NF), lambda i: (i, 0)),
        out_shape=jax.ShapeDtypeStruct((e1 - e0, NF), jnp.float32),
    )(ea_t, W_m1, b_m1.reshape(1, NF), W_m2, b_m2.reshape(1, NF))


def _sc_gather_scatter(h1, m_e, c2d, ei3, init=None):
    N, HC = h1.shape
    nrows = ei3.shape[1]
    rpw = nrows // NW
    nextra = nrows - rpw * NW
    npad = ((N + 8 * NS - 1) // (8 * NS)) * (8 * NS)
    rps = npad // NS

    mesh = plsc.VectorSubcoreMesh(core_axis_name="c", subcore_axis_name="s")
    cp = pltpu.CompilerParams()
    if "needs_layout_passes" in pltpu.CompilerParams.__dataclass_fields__:
        cp = dataclasses.replace(cp, needs_layout_passes=False)

    DI = 3
    DG = 2
    STEP = 6
    CB = 64
    nslot = 2 * rpw
    assert nslot % STEP == 0

    @functools.partial(
        pl.kernel,
        out_type=jax.ShapeDtypeStruct((NC, npad, HC), jnp.float32),
        mesh=mesh,
        compiler_params=cp,
        scratch_types=(
            [pltpu.VMEM((CB,), jnp.int32) for _ in range(DI)]
            + [pltpu.VMEM((CB,), jnp.int32) for _ in range(DI)]
            + [pltpu.VMEM((CB,), jnp.float32) for _ in range(DI)]
            + [pltpu.VMEM((CB, HC), jnp.float32) for _ in range(DI)]
            + [pltpu.VMEM((CB, HC), jnp.float32) for _ in range(DG)]
            + [pltpu.SemaphoreType.DMA for _ in range(2 * DI + 2 * DG)]
            + [pltpu.VMEM_SHARED((npad, HC), jnp.float32)]
        ),
    )
    def k(init_hbm, h1_hbm, me_hbm, c_hbm, ei_hbm, out_hbm, *scr):
        sidx = scr[0:DI]
        didx = scr[DI:2 * DI]
        c_v = scr[2 * DI:3 * DI]
        w_v = scr[3 * DI:4 * DI]
        rows = scr[4 * DI:4 * DI + DG]
        sems = scr[4 * DI + DG:4 * DI + DG + 2 * DI + 2 * DG]
        sem_i = sems[0:DI]
        sem_w = sems[DI:2 * DI]
        sem_g = sems[2 * DI:2 * DI + DG]
        sem_s = sems[2 * DI + DG:2 * DI + 2 * DG]
        agg_sh = scr[-1]

        c = lax.axis_index("c")
        s = lax.axis_index("s")
        wid = s * NC + c

        if init is None:
            pltpu.sync_copy(init_hbm, agg_sh.at[pl.ds(s * rps, rps)])
        else:
            pltpu.sync_copy(init_hbm.at[c, pl.ds(s * rps, rps)],
                            agg_sh.at[pl.ds(s * rps, rps)])
        plsc.subcore_barrier()

        base2 = wid * nslot

        def start_loads(t, m):
            row = (base2 + t) // 2
            off = (base2 + t) % 2 * CB
            pltpu.async_copy(ei_hbm.at[1, row, pl.ds(off, CB)], sidx[m],
                             sem_i[m])
            pltpu.async_copy(ei_hbm.at[0, row, pl.ds(off, CB)], didx[m],
                             sem_i[m])
            pltpu.async_copy(me_hbm.at[pl.ds(row * CK + off, CB)], w_v[m],
                             sem_w[m])
            pltpu.async_copy(c_hbm.at[row, pl.ds(off, CB)], c_v[m], sem_w[m])

        def wait_loads_idx(m):
            pltpu.make_async_copy(ei_hbm.at[1, 0, pl.ds(0, CB)], sidx[m],
                                  sem_i[m]).wait()
            pltpu.make_async_copy(ei_hbm.at[0, 0, pl.ds(0, CB)], didx[m],
                                  sem_i[m]).wait()

        def wait_loads_wc(m):
            pltpu.make_async_copy(me_hbm.at[pl.ds(0, CB)], w_v[m],
                                  sem_w[m]).wait()
            pltpu.make_async_copy(c_hbm.at[0, pl.ds(0, CB)], c_v[m],
                                  sem_w[m]).wait()

        def start_gather(m, b):
            pltpu.async_copy(h1_hbm.at[sidx[m]], rows[b], sem_g[b])

        def wait_gather(b):
            pltpu.make_async_copy(h1_hbm.at[pl.ds(0, CB)], rows[b],
                                  sem_g[b]).wait()

        def start_scatter(m, b):
            pltpu.async_copy(rows[b], agg_sh.at[didx[m]], sem_s[b],
                             add=True)

        def wait_scatter(b):
            pltpu.make_async_copy(rows[b], agg_sh.at[pl.ds(0, CB)],
                                  sem_s[b]).wait()

        def mult(m, b):
            @plsc.parallel_loop(0, CB, step=1, unroll=8)
            def _(r):
                cs = plsc.load_gather(
                    c_v[m], [jnp.full((LANES,), r, jnp.int32)])
                for cc in range(0, HC, LANES):
                    sl = (r, pl.ds(cc, LANES))
                    rows[b].at[sl][...] = (rows[b].at[sl][...]
                                           * (w_v[m].at[sl][...] * cs))

        start_loads(0, 0)
        start_loads(1, 1)
        wait_loads_idx(0)
        start_gather(0, 0)

        @pl.loop(0, nslot, step=STEP)
        def _(g):
            for j in range(STEP):
                b = j % DG
                m = j % DI
                nb = (j + 1) % DG
                nm = (j + 1) % DI
                i = g + j

                @pl.when(i + 1 < nslot)
                def _():
                    wait_loads_idx(nm)

                    @pl.when(i >= 1)
                    def _():
                        wait_scatter(nb)

                    start_gather(nm, nb)

                @pl.when(i + 2 < nslot)
                def _():
                    start_loads(i + 2, (j + 2) % DI)

                wait_gather(b)
                wait_loads_wc(m)
                mult(m, b)
                start_scatter(m, b)

        wait_scatter(0)
        wait_scatter(1)

        if nextra:
            @pl.when(wid < nextra)
            def _():
                row = NW * rpw + wid
                for h in range(2):
                    off = h * CB
                    pltpu.sync_copy(ei_hbm.at[1, row, pl.ds(off, CB)],
                                    sidx[0])
                    pltpu.sync_copy(ei_hbm.at[0, row, pl.ds(off, CB)],
                                    didx[0])
                    pltpu.sync_copy(h1_hbm.at[sidx[0]], rows[0])
                    pltpu.sync_copy(me_hbm.at[pl.ds(row * CK + off, CB)],
                                    w_v[0])
                    pltpu.sync_copy(c_hbm.at[row, pl.ds(off, CB)], c_v[0])
                    mult(0, 0)
                    pltpu.sync_copy(rows[0], agg_sh.at[didx[0]], add=True)

        plsc.subcore_barrier()
        pltpu.sync_copy(agg_sh.at[pl.ds(s * rps, rps)],
                        out_hbm.at[c, pl.ds(s * rps, rps)])

    first = (jnp.zeros((rps, HC), jnp.float32) if init is None else init)
    return k(first, h1, m_e, c2d, ei3)


def _tail(partials, W_lin2, b_lin2, W_out, b_out):
    _, N, NF = partials.shape
    HC = W_lin2.shape[1]

    def body(p_ref, w1_ref, b1_ref, w2_ref, b2_ref, o_ref):
        agg = p_ref[0] + p_ref[1]
        h2 = jnp.dot(agg, w1_ref[...],
                     preferred_element_type=jnp.float32) + b1_ref[...]
        h3 = _sp(h2)
        o_ref[...] = jnp.dot(h3, w2_ref[...],
                             preferred_element_type=jnp.float32) + b2_ref[...]

    return pl.pallas_call(
        body,
        out_shape=jax.ShapeDtypeStruct((N, HC), jnp.float32),
    )(partials, W_lin2, b_lin2.reshape(1, HC), W_out, b_out.reshape(1, HC))


def kernel(h, edge_index, edge_weight, edge_attr,
           W_lin1, W_m1, b_m1, W_m2, b_m2, W_lin2, b_lin2, W_out, b_out):
    N = h.shape[0]
    E = edge_weight.shape[0]
    Eh = E // 2
    rh = Eh // CK
    ei3 = _repack_idx(edge_index.astype(jnp.int32))
    ew2d = edge_weight.reshape(E // CK, CK)

    h1 = _node_matmul(h, W_lin1)
    c2d = _cutoff(ew2d)
    m_1 = _edge_mlp(edge_attr.T, W_m1, b_m1, W_m2, b_m2, (0, Eh))
    m_2 = _edge_mlp(edge_attr.T, W_m1, b_m1, W_m2, b_m2, (Eh, E))
    p_1 = _sc_gather_scatter(h1, m_1, c2d[:rh], ei3[:, :rh])
    p_2 = _sc_gather_scatter(h1, m_2, c2d[rh:], ei3[:, rh:], init=p_1)
    return _tail(p_2, W_lin2, b_lin2, W_out, b_out)[:N]


# speedup vs baseline: 5.1543x; 1.0059x over previous
import dataclasses
import functools
import math
from math import pi as PI

import jax
import jax.numpy as jnp
from jax import lax
from jax.experimental import pallas as pl
from jax.experimental.pallas import tpu as pltpu
from jax.experimental.pallas import tpu_sc as plsc

CUTOFF = 10.0
SHIFT = math.log(2.0)

NC = 2
NS = 16
LANES = 16
NW = NC * NS

CK = 128


def _sp(x):
    return jnp.where(x > 20.0, x - SHIFT, jnp.log(0.5 + 0.5 * jnp.exp(x)))


def _node_matmul(h, w):
    n, _ = h.shape
    m = w.shape[1]

    def body(h_ref, w_ref, o_ref):
        o_ref[...] = jnp.dot(h_ref[...], w_ref[...],
                             preferred_element_type=jnp.float32)

    return pl.pallas_call(
        body,
        out_shape=jax.ShapeDtypeStruct((n, m), jnp.float32),
    )(h, w)


def _repack_idx(edge_index):
    E = edge_index.shape[1]

    def body(ei_ref, o_ref):
        o_ref[...] = ei_ref[...].reshape(2, E // CK, CK)

    return pl.pallas_call(
        body,
        out_shape=jax.ShapeDtypeStruct((2, E // CK, CK), jnp.int32),
    )(edge_index)


def _cutoff(ew2d):
    def body(ew_ref, c_ref):
        c_ref[...] = 0.5 * (jnp.cos(ew_ref[...] * (PI / CUTOFF)) + 1.0)

    return pl.pallas_call(
        body,
        out_shape=jax.ShapeDtypeStruct(ew2d.shape, jnp.float32),
    )(ew2d)


def _edge_mlp(ea_t, W_m1, b_m1, W_m2, b_m2, erange):
    NG, E = ea_t.shape
    NF = W_m1.shape[1]
    BE = 3200
    e0, e1 = erange
    grid = ((e1 - e0) // BE,)
    blk0 = e0 // BE

    def body(ea_ref, w1_ref, b1_ref, w2_ref, b2_ref, o_ref):
        hid = lax.dot_general(
            ea_ref[...], w1_ref[...], (((0,), (0,)), ((), ())),
            preferred_element_type=jnp.float32) + b1_ref[...]
        hid = _sp(hid)
        o_ref[...] = jnp.dot(hid, w2_ref[...],
                             preferred_element_type=jnp.float32) + b2_ref[...]

    return pl.pallas_call(
        body,
        grid=grid,
        in_specs=[
            pl.BlockSpec((NG, BE), lambda i: (0, i + blk0)),
            pl.BlockSpec((NG, NF), lambda i: (0, 0)),
            pl.BlockSpec((1, NF), lambda i: (0, 0)),
            pl.BlockSpec((NF, NF), lambda i: (0, 0)),
            pl.BlockSpec((1, NF), lambda i: (0, 0)),
        ],
        out_specs=pl.BlockSpec((BE, NF), lambda i: (i, 0)),
        out_shape=jax.ShapeDtypeStruct((e1 - e0, NF), jnp.float32),
    )(ea_t, W_m1, b_m1.reshape(1, NF), W_m2, b_m2.reshape(1, NF))


def _sc_gather_scatter(h1, m_e, c2d, ei3, init=None):
    N, HC = h1.shape
    nrows = ei3.shape[1]
    rpw = nrows // NW
    nextra = nrows - rpw * NW
    rps = (N // NS) // 8 * 8
    rpl = N - rps * (NS - 1)

    mesh = plsc.VectorSubcoreMesh(core_axis_name="c", subcore_axis_name="s")
    cp = pltpu.CompilerParams()
    if "needs_layout_passes" in pltpu.CompilerParams.__dataclass_fields__:
        cp = dataclasses.replace(cp, needs_layout_passes=False)

    DI = 3
    DG = 2
    STEP = 6
    CB = 64
    nslot = 2 * rpw
    assert nslot % STEP == 0

    @functools.partial(
        pl.kernel,
        out_type=jax.ShapeDtypeStruct((NC, N, HC), jnp.float32),
        mesh=mesh,
        compiler_params=cp,
        scratch_types=(
            [pltpu.VMEM((CB,), jnp.int32) for _ in range(DI)]
            + [pltpu.VMEM((CB,), jnp.int32) for _ in range(DI)]
            + [pltpu.VMEM((CB,), jnp.float32) for _ in range(DI)]
            + [pltpu.VMEM((CB, HC), jnp.float32) for _ in range(DI)]
            + [pltpu.VMEM((CB, HC), jnp.float32) for _ in range(DG)]
            + [pltpu.SemaphoreType.DMA for _ in range(2 * DI + 2 * DG)]
            + [pltpu.VMEM_SHARED((N, HC), jnp.float32)]
        ),
    )
    def k(init_hbm, h1_hbm, me_hbm, c_hbm, ei_hbm, out_hbm, *scr):
        sidx = scr[0:DI]
        didx = scr[DI:2 * DI]
        c_v = scr[2 * DI:3 * DI]
        w_v = scr[3 * DI:4 * DI]
        rows = scr[4 * DI:4 * DI + DG]
        sems = scr[4 * DI + DG:4 * DI + DG + 2 * DI + 2 * DG]
        sem_i = sems[0:DI]
        sem_w = sems[DI:2 * DI]
        sem_g = sems[2 * DI:2 * DI + DG]
        sem_s = sems[2 * DI + DG:2 * DI + 2 * DG]
        agg_sh = scr[-1]

        c = lax.axis_index("c")
        s = lax.axis_index("s")
        wid = s * NC + c

        def init_readout(sz, off, readout):
            sl = pl.ds(off, sz)
            if readout:
                pltpu.sync_copy(agg_sh.at[sl], out_hbm.at[c, sl])
            elif init is None:
                pltpu.sync_copy(init_hbm.at[pl.ds(0, sz)], agg_sh.at[sl])
            else:
                pltpu.sync_copy(init_hbm.at[c, sl], agg_sh.at[sl])

        def each_range(readout):
            @pl.when(s < NS - 1)
            def _():
                init_readout(rps, s * rps, readout)

            @pl.when(s == NS - 1)
            def _():
                init_readout(rpl, (NS - 1) * rps, readout)

        each_range(False)
        plsc.subcore_barrier()

        base2 = wid * nslot

        def start_loads(t, m):
            row = (base2 + t) // 2
            off = (base2 + t) % 2 * CB
            pltpu.async_copy(ei_hbm.at[1, row, pl.ds(off, CB)], sidx[m],
                             sem_i[m])
            pltpu.async_copy(ei_hbm.at[0, row, pl.ds(off, CB)], didx[m],
                             sem_i[m])
            pltpu.async_copy(me_hbm.at[pl.ds(row * CK + off, CB)], w_v[m],
                             sem_w[m])
            pltpu.async_copy(c_hbm.at[row, pl.ds(off, CB)], c_v[m], sem_w[m])

        def wait_loads_idx(m):
            pltpu.make_async_copy(ei_hbm.at[1, 0, pl.ds(0, CB)], sidx[m],
                                  sem_i[m]).wait()
            pltpu.make_async_copy(ei_hbm.at[0, 0, pl.ds(0, CB)], didx[m],
                                  sem_i[m]).wait()

        def wait_loads_wc(m):
            pltpu.make_async_copy(me_hbm.at[pl.ds(0, CB)], w_v[m],
                                  sem_w[m]).wait()
            pltpu.make_async_copy(c_hbm.at[0, pl.ds(0, CB)], c_v[m],
                                  sem_w[m]).wait()

        def start_gather(m, b):
            pltpu.async_copy(h1_hbm.at[sidx[m]], rows[b], sem_g[b])

        def wait_gather(b):
            pltpu.make_async_copy(h1_hbm.at[pl.ds(0, CB)], rows[b],
                                  sem_g[b]).wait()

        def start_scatter(m, b):
            pltpu.async_copy(rows[b], agg_sh.at[didx[m]], sem_s[b],
                             add=True)

        def wait_scatter(b):
            pltpu.make_async_copy(rows[b], agg_sh.at[pl.ds(0, CB)],
                                  sem_s[b]).wait()

        def mult(m, b):
            @plsc.parallel_loop(0, CB, step=1, unroll=8)
            def _(r):
                cs = plsc.load_gather(
                    c_v[m], [jnp.full((LANES,), r, jnp.int32)])
                for cc in range(0, HC, LANES):
                    sl = (r, pl.ds(cc, LANES))
                    rows[b].at[sl][...] = (rows[b].at[sl][...]
                                           * (w_v[m].at[sl][...] * cs))

        start_loads(0, 0)
        start_loads(1, 1)
        wait_loads_idx(0)
        start_gather(0, 0)

        @pl.loop(0, nslot, step=STEP)
        def _(g):
            for j in range(STEP):
                b = j % DG
                m = j % DI
                nb = (j + 1) % DG
                nm = (j + 1) % DI
                i = g + j

                @pl.when(i + 1 < nslot)
                def _():
                    wait_loads_idx(nm)

                    @pl.when(i >= DG - 1)
                    def _():
                        wait_scatter(nb)

                    start_gather(nm, nb)

                @pl.when(i + 2 < nslot)
                def _():
                    start_loads(i + 2, (j + 2) % DI)

                wait_gather(b)
                wait_loads_wc(m)
                mult(m, b)
                start_scatter(m, b)

        for d in range(DG):
            wait_scatter(d)

        if nextra:
            @pl.when(wid < nextra)
            def _():
                row = NW * rpw + wid
                for h in range(2):
                    off = h * CB
                    pltpu.sync_copy(ei_hbm.at[1, row, pl.ds(off, CB)],
                                    sidx[0])
                    pltpu.sync_copy(ei_hbm.at[0, row, pl.ds(off, CB)],
                                    didx[0])
                    pltpu.sync_copy(h1_hbm.at[sidx[0]], rows[0])
                    pltpu.sync_copy(me_hbm.at[pl.ds(row * CK + off, CB)],
                                    w_v[0])
                    pltpu.sync_copy(c_hbm.at[row, pl.ds(off, CB)], c_v[0])
                    mult(0, 0)
                    pltpu.sync_copy(rows[0], agg_sh.at[didx[0]], add=True)

        plsc.subcore_barrier()
        each_range(True)

    first = (jnp.zeros((rpl, HC), jnp.float32) if init is None else init)
    return k(first, h1, m_e, c2d, ei3)


def _tail(partials, W_lin2, b_lin2, W_out, b_out):
    _, N, NF = partials.shape
    HC = W_lin2.shape[1]

    def body(p_ref, w1_ref, b1_ref, w2_ref, b2_ref, o_ref):
        agg = p_ref[0] + p_ref[1]
        h2 = jnp.dot(agg, w1_ref[...],
                     preferred_element_type=jnp.float32) + b1_ref[...]
        h3 = _sp(h2)
        o_ref[...] = jnp.dot(h3, w2_ref[...],
                             preferred_element_type=jnp.float32) + b2_ref[...]

    return pl.pallas_call(
        body,
        out_shape=jax.ShapeDtypeStruct((N, HC), jnp.float32),
    )(partials, W_lin2, b_lin2.reshape(1, HC), W_out, b_out.reshape(1, HC))


def kernel(h, edge_index, edge_weight, edge_attr,
           W_lin1, W_m1, b_m1, W_m2, b_m2, W_lin2, b_lin2, W_out, b_out):
    N = h.shape[0]
    E = edge_weight.shape[0]
    Eh = E // 2
    rh = Eh // CK
    ei3 = _repack_idx(edge_index.astype(jnp.int32))
    ew2d = edge_weight.reshape(E // CK, CK)

    h1 = _node_matmul(h, W_lin1)
    c2d = _cutoff(ew2d)
    m_1 = _edge_mlp(edge_attr.T, W_m1, b_m1, W_m2, b_m2, (0, Eh))
    m_2 = _edge_mlp(edge_attr.T, W_m1, b_m1, W_m2, b_m2, (Eh, E))
    p_1 = _sc_gather_scatter(h1, m_1, c2d[:rh], ei3[:, :rh])
    p_2 = _sc_gather_scatter(h1, m_2, c2d[rh:], ei3[:, rh:], init=p_1)
    return _tail(p_2, W_lin2, b_lin2, W_out, b_out)[:N]
